# baseline (speedup 1.0000x reference)
import functools

import jax
import jax.numpy as jnp
from jax import lax
from jax.experimental import pallas as pl
from jax.experimental.pallas import tpu as pltpu
from jax.experimental.pallas import tpu_sc as plsc

NUM_PREPEND = 16
CHUNK_ROWS = 32


def kernel(x, embed_table):
    B, S, D = x.shape
    SO = S + NUM_PREPEND
    info = plsc.get_sparse_core_info()
    NW = info.num_cores * info.num_subcores
    rows_per_w = (B * S) // NW
    w_per_batch = S // rows_per_w
    nch = rows_per_w // CHUNK_ROWS
    mesh = plsc.VectorSubcoreMesh(core_axis_name="c", subcore_axis_name="s")

    @functools.partial(
        pl.kernel,
        mesh=mesh,
        out_type=jax.ShapeDtypeStruct((B * SO, D), x.dtype),
        scratch_types=[
            pltpu.VMEM((2, CHUNK_ROWS, D), x.dtype),
            pltpu.VMEM((NUM_PREPEND, D), x.dtype),
            pltpu.SemaphoreType.DMA((2,)),
            pltpu.SemaphoreType.DMA((2,)),
            pltpu.SemaphoreType.DMA,
        ],
    )
    def sc_copy(x_hbm, emb_hbm, out_hbm, buf, emb_v, ld, st, es):
        wid = lax.axis_index("s") * info.num_cores + lax.axis_index("c")
        b = wid // w_per_batch
        c = wid % w_per_batch
        src_base = wid * rows_per_w
        dst_base = b * SO + NUM_PREPEND + c * rows_per_w

        loads = []
        stores = []
        for t in range(nch):
            j = t % 2
            loads.append(pltpu.make_async_copy(
                x_hbm.at[pl.ds(src_base + t * CHUNK_ROWS, CHUNK_ROWS)],
                buf.at[j], ld.at[j]))
            stores.append(pltpu.make_async_copy(
                buf.at[j],
                out_hbm.at[pl.ds(dst_base + t * CHUNK_ROWS, CHUNK_ROWS)],
                st.at[j]))

        emb_copy = pltpu.make_async_copy(emb_hbm, emb_v, es)
        prefix_copy = pltpu.make_async_copy(
            emb_v, out_hbm.at[pl.ds(wid * SO, NUM_PREPEND)], es)

        @pl.when(wid < B)
        def _():
            emb_copy.start()

        for t in range(nch + 1):
            if t < nch:
                if t >= 2:
                    stores[t - 2].wait()
                loads[t].start()
            if t == 0:
                @pl.when(wid < B)
                def _():
                    emb_copy.wait()
                    prefix_copy.start()
            if t >= 1:
                loads[t - 1].wait()
                stores[t - 1].start()

        stores[nch - 2].wait()
        stores[nch - 1].wait()

        @pl.when(wid < B)
        def _():
            prefix_copy.wait()

    out = sc_copy(x.reshape(B * S, D), embed_table)
    return out.reshape(B, SO, D)


# speedup vs baseline: 1.0033x; 1.0033x over previous
import functools

import jax
import jax.numpy as jnp
from jax import lax
from jax.experimental import pallas as pl
from jax.experimental.pallas import tpu as pltpu
from jax.experimental.pallas import tpu_sc as plsc

NUM_PREPEND = 16
CHUNK_ROWS = 32
NBUF = 3


def kernel(x, embed_table):
    B, S, D = x.shape
    SO = S + NUM_PREPEND
    info = plsc.get_sparse_core_info()
    NW = info.num_cores * info.num_subcores
    rows_per_w = (B * S) // NW
    w_per_batch = S // rows_per_w
    nch = rows_per_w // CHUNK_ROWS
    mesh = plsc.VectorSubcoreMesh(core_axis_name="c", subcore_axis_name="s")

    @functools.partial(
        pl.kernel,
        mesh=mesh,
        out_type=jax.ShapeDtypeStruct((B * SO, D), x.dtype),
        scratch_types=[
            pltpu.VMEM((NBUF, CHUNK_ROWS, D), x.dtype),
            pltpu.VMEM((NUM_PREPEND, D), x.dtype),
            pltpu.SemaphoreType.DMA((NBUF,)),
            pltpu.SemaphoreType.DMA((NBUF,)),
            pltpu.SemaphoreType.DMA,
        ],
    )
    def sc_copy(x_hbm, emb_hbm, out_hbm, buf, emb_v, ld, st, es):
        wid = lax.axis_index("s") * info.num_cores + lax.axis_index("c")
        b = wid // w_per_batch
        c = wid % w_per_batch
        src_base = wid * rows_per_w
        dst_base = b * SO + NUM_PREPEND + c * rows_per_w

        loads = []
        stores = []
        for t in range(nch):
            j = t % NBUF
            loads.append(pltpu.make_async_copy(
                x_hbm.at[pl.ds(src_base + t * CHUNK_ROWS, CHUNK_ROWS)],
                buf.at[j], ld.at[j]))
            stores.append(pltpu.make_async_copy(
                buf.at[j],
                out_hbm.at[pl.ds(dst_base + t * CHUNK_ROWS, CHUNK_ROWS)],
                st.at[j]))

        emb_copy = pltpu.make_async_copy(emb_hbm, emb_v, es)
        prefix_copy = pltpu.make_async_copy(
            emb_v, out_hbm.at[pl.ds(wid * SO, NUM_PREPEND)], es)

        @pl.when(wid < B)
        def _():
            emb_copy.start()

        lag = NBUF - 1
        for t in range(nch + lag):
            if t < nch:
                if t >= NBUF:
                    stores[t - NBUF].wait()
                loads[t].start()
            if t == 0:
                @pl.when(wid < B)
                def _():
                    emb_copy.wait()
                    prefix_copy.start()
            k = t - lag
            if 0 <= k < nch:
                loads[k].wait()
                stores[k].start()

        for t in range(nch - NBUF, nch):
            stores[t].wait()

        @pl.when(wid < B)
        def _():
            prefix_copy.wait()

    out = sc_copy(x.reshape(B * S, D), embed_table)
    return out.reshape(B, SO, D)


# speedup vs baseline: 1.5694x; 1.5642x over previous
import jax
import jax.numpy as jnp
from jax.experimental import pallas as pl
from jax.experimental.pallas import tpu as pltpu

NUM_PREPEND = 16
CHUNK_ROWS = 1024
NBUF = 8
DEPTH = 6


def _prepend_body(x_hbm, emb_hbm, out_hbm, buf, emb_v,
                  ld_sems, st_sems, esem, tsems):
    B, S, D = x_hbm.shape
    per_batch = S // CHUNK_ROWS
    nch = B * per_batch

    emb_load = pltpu.make_async_copy(emb_hbm, emb_v, esem)
    emb_load.start()

    loads = []
    stores = []
    for i in range(nch):
        b, c = divmod(i, per_batch)
        j = i % NBUF
        loads.append(pltpu.make_async_copy(
            x_hbm.at[b, pl.ds(c * CHUNK_ROWS, CHUNK_ROWS)],
            buf.at[j], ld_sems.at[j]))
        stores.append(pltpu.make_async_copy(
            buf.at[j],
            out_hbm.at[b, pl.ds(NUM_PREPEND + c * CHUNK_ROWS, CHUNK_ROWS)],
            st_sems.at[j]))

    for i in range(nch + DEPTH):
        if i < nch:
            if i >= NBUF:
                stores[i - NBUF].wait()
            loads[i].start()
        if i == 0:
            emb_load.wait()
            for b in range(B):
                pltpu.make_async_copy(
                    emb_v, out_hbm.at[b, pl.ds(0, NUM_PREPEND)], tsems.at[b]
                ).start()
        k = i - DEPTH
        if 0 <= k < nch:
            loads[k].wait()
            stores[k].start()

    for i in range(nch - NBUF, nch):
        stores[i].wait()
    for b in range(B):
        pltpu.make_async_copy(
            emb_v, out_hbm.at[b, pl.ds(0, NUM_PREPEND)], tsems.at[b]
        ).wait()


def kernel(x, embed_table):
    B, S, D = x.shape
    out_shape = jax.ShapeDtypeStruct((B, S + NUM_PREPEND, D), x.dtype)
    return pl.pallas_call(
        _prepend_body,
        out_shape=out_shape,
        in_specs=[
            pl.BlockSpec(memory_space=pltpu.MemorySpace.HBM),
            pl.BlockSpec(memory_space=pltpu.MemorySpace.HBM),
        ],
        out_specs=pl.BlockSpec(memory_space=pltpu.MemorySpace.HBM),
        scratch_shapes=[
            pltpu.VMEM((NBUF, CHUNK_ROWS, D), x.dtype),
            pltpu.VMEM((NUM_PREPEND, D), embed_table.dtype),
            pltpu.SemaphoreType.DMA((NBUF,)),
            pltpu.SemaphoreType.DMA((NBUF,)),
            pltpu.SemaphoreType.DMA,
            pltpu.SemaphoreType.DMA((B,)),
        ],
    )(x, embed_table)


# speedup vs baseline: 1.5737x; 1.0027x over previous
import jax
import jax.numpy as jnp
from jax.experimental import pallas as pl
from jax.experimental.pallas import tpu as pltpu

NUM_PREPEND = 16
CHUNK_ROWS = 2048
NBUF = 6
DEPTH = 4


def _prepend_body(x_hbm, emb_hbm, out_hbm, buf, emb_v,
                  ld_sems, st_sems, esem, tsems):
    B, S, D = x_hbm.shape
    per_batch = S // CHUNK_ROWS
    nch = B * per_batch

    emb_load = pltpu.make_async_copy(emb_hbm, emb_v, esem)
    emb_load.start()

    loads = []
    stores = []
    for i in range(nch):
        b, c = divmod(i, per_batch)
        j = i % NBUF
        loads.append(pltpu.make_async_copy(
            x_hbm.at[b, pl.ds(c * CHUNK_ROWS, CHUNK_ROWS)],
            buf.at[j], ld_sems.at[j]))
        stores.append(pltpu.make_async_copy(
            buf.at[j],
            out_hbm.at[b, pl.ds(NUM_PREPEND + c * CHUNK_ROWS, CHUNK_ROWS)],
            st_sems.at[j]))

    for i in range(nch + DEPTH):
        if i < nch:
            if i >= NBUF:
                stores[i - NBUF].wait()
            loads[i].start()
        if i == 0:
            emb_load.wait()
            for b in range(B):
                pltpu.make_async_copy(
                    emb_v, out_hbm.at[b, pl.ds(0, NUM_PREPEND)], tsems.at[b]
                ).start()
        k = i - DEPTH
        if 0 <= k < nch:
            loads[k].wait()
            stores[k].start()

    for i in range(nch - NBUF, nch):
        stores[i].wait()
    for b in range(B):
        pltpu.make_async_copy(
            emb_v, out_hbm.at[b, pl.ds(0, NUM_PREPEND)], tsems.at[b]
        ).wait()


def kernel(x, embed_table):
    B, S, D = x.shape
    out_shape = jax.ShapeDtypeStruct((B, S + NUM_PREPEND, D), x.dtype)
    return pl.pallas_call(
        _prepend_body,
        out_shape=out_shape,
        in_specs=[
            pl.BlockSpec(memory_space=pltpu.MemorySpace.HBM),
            pl.BlockSpec(memory_space=pltpu.MemorySpace.HBM),
        ],
        out_specs=pl.BlockSpec(memory_space=pltpu.MemorySpace.HBM),
        scratch_shapes=[
            pltpu.VMEM((NBUF, CHUNK_ROWS, D), x.dtype),
            pltpu.VMEM((NUM_PREPEND, D), embed_table.dtype),
            pltpu.SemaphoreType.DMA((NBUF,)),
            pltpu.SemaphoreType.DMA((NBUF,)),
            pltpu.SemaphoreType.DMA,
            pltpu.SemaphoreType.DMA((B,)),
        ],
    )(x, embed_table)
